# baseline (speedup 1.0000x reference)
import functools

import jax
import jax.numpy as jnp
from jax import lax
from jax.experimental import pallas as pl
from jax.experimental.pallas import tpu as pltpu
from jax.experimental.pallas import tpu_sc as plsc

_B, _L = 4096, 50
_N = _B * _L
_LOC_DIM, _TIM_DIM, _ACT_DIM = 64, 32, 32
_LOC_W = _LOC_DIM // 2
_TIM_W = _TIM_DIM // 2
_ACT_W = _ACT_DIM // 2
_HID = 256

_NC, _NS = 2, 16
_NW = _NC * _NS
_RPW = _N // _NW
_CH = 128
_NCH = _RPW // _CH
_GPB = 5
_RPB = _CH * _GPB
_NB = _RPW // _RPB

_BK = 8192


def _sc_body(xl_hbm, xt_hbm, xa_hbm, loc_hbm, tim_hbm, act_hbm,
             lout, tout, aout, il, it, ia,
             rl0, rt0, ra0, rl1, rt1, ra1, sem0, sem1):
    wid = lax.axis_index("s") * _NC + lax.axis_index("c")
    pltpu.sync_copy(xl_hbm.at[wid], il)
    pltpu.sync_copy(xt_hbm.at[wid], it)
    pltpu.sync_copy(xa_hbm.at[wid], ia)

    bufs = ((rl0, rt0, ra0, sem0), (rl1, rt1, ra1, sem1))

    def issue(b, which):
        rl, rt, ra, sem = bufs[which]
        for j in range(_GPB):
            ci = _GPB * b + j
            s = pl.ds(j * _CH, _CH)
            pltpu.async_copy(loc_hbm.at[il.at[ci]], rl.at[s], sem)
            pltpu.async_copy(tim_hbm.at[it.at[ci]], rt.at[s], sem)
            pltpu.async_copy(act_hbm.at[ia.at[ci]], ra.at[s], sem)

    def drain_and_writeback(b, which):
        rl, rt, ra, sem = bufs[which]
        base = wid * _RPW + b * _RPB
        for buf, out in ((rl, lout), (rt, tout), (ra, aout)):
            pltpu.make_async_copy(out.at[pl.ds(base, _RPB)], buf, sem).wait()
        for buf, out in ((rl, lout), (rt, tout), (ra, aout)):
            pltpu.sync_copy(buf, out.at[pl.ds(base, _RPB)])

    issue(0, 0)
    for b in range(1, _NB):
        issue(b, b % 2)
        drain_and_writeback(b - 1, (b - 1) % 2)
    drain_and_writeback(_NB - 1, (_NB - 1) % 2)


def _sc_gather(xl, xt, xa, loc_packed, tim_packed, act_packed):
    mesh = plsc.VectorSubcoreMesh(core_axis_name="c", subcore_axis_name="s")
    kern = pl.kernel(
        _sc_body,
        out_type=(
            jax.ShapeDtypeStruct((_N, _LOC_W), jnp.int32),
            jax.ShapeDtypeStruct((_N, _TIM_W), jnp.int32),
            jax.ShapeDtypeStruct((_N, _ACT_W), jnp.int32),
        ),
        mesh=mesh,
        scratch_types=[
            pltpu.VMEM((_NCH, _CH), jnp.int32),
            pltpu.VMEM((_NCH, _CH), jnp.int32),
            pltpu.VMEM((_NCH, _CH), jnp.int32),
            pltpu.VMEM((_RPB, _LOC_W), jnp.int32),
            pltpu.VMEM((_RPB, _TIM_W), jnp.int32),
            pltpu.VMEM((_RPB, _ACT_W), jnp.int32),
            pltpu.VMEM((_RPB, _LOC_W), jnp.int32),
            pltpu.VMEM((_RPB, _TIM_W), jnp.int32),
            pltpu.VMEM((_RPB, _ACT_W), jnp.int32),
            pltpu.SemaphoreType.DMA,
            pltpu.SemaphoreType.DMA,
        ],
        compiler_params=pltpu.CompilerParams(use_tc_tiling_on_sc=False),
    )
    return kern(xl, xt, xa, loc_packed, tim_packed, act_packed)


def _unpack(words):
    f32 = jnp.float32
    bf = jnp.bfloat16
    lo = lax.bitcast_convert_type(jnp.left_shift(words, 16), f32).astype(bf)
    hi = lax.bitcast_convert_type(
        jnp.bitwise_and(words, jnp.int32(-65536)), f32).astype(bf)
    return lo, hi


def _mlp_body(lp_ref, tp_ref, ap_ref, w1, b1, w2, b2, w3, b3,
              w4, b4, w5, b5, o_ref):
    f32 = jnp.float32
    bf = jnp.bfloat16
    l_lo, l_hi = _unpack(lp_ref[...])
    t_lo, t_hi = _unpack(tp_ref[...])
    a_lo, a_hi = _unpack(ap_ref[...])
    x = jnp.concatenate([l_lo, l_hi, t_lo, t_hi, a_lo, a_hi], axis=1)
    h = jnp.dot(x, w1[...], preferred_element_type=f32)
    h = jnp.maximum(h.astype(bf) + b1[...], 0)
    h = jnp.maximum(jnp.dot(h, w2[...], preferred_element_type=f32).astype(bf) + b2[...], 0)
    h = jnp.maximum(jnp.dot(h, w3[...], preferred_element_type=f32).astype(bf) + b3[...], 0)
    h = jnp.maximum(jnp.dot(h, w4[...], preferred_element_type=f32) + b4[...], 0.0)
    z = jnp.sum(h.reshape(_BK // 128, 128, _HID) * w5[...], axis=2) + b5[...]
    o_ref[...] = 1.0 / (1.0 + jnp.exp(-z))


def _mlp(lemb, temb, aemb, W1, b1, W2, b2, W3, b3, W4, b4, w5t, b5):
    full = lambda shape: pl.BlockSpec(shape, lambda i: tuple(0 for _ in shape))
    return pl.pallas_call(
        _mlp_body,
        grid=(_N // _BK,),
        in_specs=[
            pl.BlockSpec((_BK, _LOC_W), lambda i: (i, 0)),
            pl.BlockSpec((_BK, _TIM_W), lambda i: (i, 0)),
            pl.BlockSpec((_BK, _ACT_W), lambda i: (i, 0)),
            full((_LOC_DIM + _TIM_DIM + _ACT_DIM, _HID)),
            full((1, _HID)),
            full((_HID, _HID)),
            full((1, _HID)),
            full((_HID, _HID)),
            full((1, _HID)),
            full((_HID, _HID)),
            full((1, _HID)),
            full((1, 1, _HID)),
            full((1, 1)),
        ],
        out_specs=pl.BlockSpec((_BK // 128, 128), lambda i: (i, 0)),
        out_shape=jax.ShapeDtypeStruct((_N // 128, 128), jnp.float32),
        compiler_params=pltpu.CompilerParams(
            dimension_semantics=("arbitrary",),
        ),
    )(lemb, temb, aemb, W1, b1, W2, b2, W3, b3, W4, b4, w5t, b5)


def _pack_bf16(table):
    rows, dim = table.shape
    return lax.bitcast_convert_type(
        table.astype(jnp.bfloat16).reshape(rows, dim // 2, 2), jnp.int32)


def _evenodd(w):
    return jnp.concatenate([w[0::2], w[1::2]], axis=0)


def kernel(x_l, x_t, x_a, loc_table, tim_table, act_table,
           W1, b1, W2, b2, W3, b3, W4, b4, W5, b5):
    bf = jnp.bfloat16
    xl = x_l.reshape(_NW, _NCH, _CH)
    xt = x_t.reshape(_NW, _NCH, _CH)
    xa = x_a.reshape(_NW, _NCH, _CH)
    lemb, temb, aemb = _sc_gather(
        xl, xt, xa, _pack_bf16(loc_table), _pack_bf16(tim_table),
        _pack_bf16(act_table))

    w1perm = jnp.concatenate(
        [_evenodd(W1[:_LOC_DIM]),
         _evenodd(W1[_LOC_DIM:_LOC_DIM + _TIM_DIM]),
         _evenodd(W1[_LOC_DIM + _TIM_DIM:])], axis=0).astype(bf)

    out = _mlp(
        lemb, temb, aemb, w1perm,
        b1.reshape(1, _HID).astype(bf), W2.astype(bf),
        b2.reshape(1, _HID).astype(bf), W3.astype(bf),
        b3.reshape(1, _HID).astype(bf), W4.astype(bf),
        b4.reshape(1, _HID), W5.reshape(1, 1, _HID), b5.reshape(1, 1),
    )
    return out.reshape(_B, _L, 1)
